# Initial kernel scaffold; baseline (speedup 1.0000x reference)
import jax
import jax.numpy as jnp
from jax.experimental import pallas as pl


def kernel(x, edge_index, W_l, b_l, W_r, gamma, beta):
    raise NotImplementedError("write your pallas kernel here")


# speedup vs baseline: 8.7901x; 8.7901x over previous
import functools

import jax
import jax.numpy as jnp
from jax import lax
from jax.experimental import pallas as pl
from jax.experimental.pallas import tpu as pltpu
from jax.experimental.pallas import tpu_sc as plsc

N = 10000
E = 320000
D = 128

NC = 2
NS = 16
NW = NC * NS

E_PER_W = E // NW
CHUNK = 80
NCHUNK = E_PER_W // CHUNK
NPAD = 10240
RPT = NPAD // NS
ZROWS = 64


def _sc_aggregate(x, src3, dst3):
  mesh = plsc.VectorSubcoreMesh(core_axis_name="c", subcore_axis_name="s")

  @functools.partial(
      pl.kernel,
      out_type=(
          jax.ShapeDtypeStruct((NC, NPAD, D), jnp.float32),
          jax.ShapeDtypeStruct((NC, NPAD), jnp.float32),
      ),
      mesh=mesh,
      compiler_params=pltpu.CompilerParams(use_tc_tiling_on_sc=False),
      scratch_types=[
          pltpu.VMEM((NCHUNK, CHUNK), jnp.int32),
          pltpu.VMEM((NCHUNK, CHUNK), jnp.int32),
          pltpu.VMEM((CHUNK, D), jnp.float32),
          pltpu.VMEM((CHUNK,), jnp.float32),
          pltpu.VMEM((ZROWS, D), jnp.float32),
          pltpu.VMEM((RPT,), jnp.float32),
          pltpu.VMEM_SHARED((NPAD, D), jnp.float32),
          pltpu.VMEM_SHARED((NPAD,), jnp.float32),
          pltpu.SemaphoreType.DMA,
      ],
  )
  def sc_kernel(x_hbm, src_hbm, dst_hbm, agg_out, cnt_out,
                srcl, dstl, rows, ones, zbuf, czbuf, agg_sh, cnt_sh, sem):
    c = lax.axis_index("c")
    s = lax.axis_index("s")
    wid = s * NC + c

    def zrow(i, _):
      zbuf[i // 8, pl.ds((i % 8) * 16, 16)] = jnp.zeros((16,), jnp.float32)
      return 0
    lax.fori_loop(0, ZROWS * (D // 16), zrow, 0)

    def czrow(i, _):
      czbuf[pl.ds(i * 16, 16)] = jnp.zeros((16,), jnp.float32)
      return 0
    lax.fori_loop(0, RPT // 16, czrow, 0)

    def onerow(i, _):
      ones[pl.ds(i * 16, 16)] = jnp.ones((16,), jnp.float32)
      return 0
    lax.fori_loop(0, CHUNK // 16, onerow, 0)

    base = s * RPT
    for k in range(RPT // ZROWS):
      pltpu.sync_copy(zbuf, agg_sh.at[pl.ds(base + k * ZROWS, ZROWS)])
    pltpu.sync_copy(czbuf, cnt_sh.at[pl.ds(base, RPT)])
    plsc.subcore_barrier()

    pltpu.sync_copy(src_hbm.at[wid], srcl)
    pltpu.sync_copy(dst_hbm.at[wid], dstl)

    def chunk_body(j, _):
      pltpu.async_copy(x_hbm.at[srcl.at[j]], rows, sem).wait()
      pltpu.sync_copy(rows, agg_sh.at[dstl.at[j]], add=True)
      pltpu.sync_copy(ones, cnt_sh.at[dstl.at[j]], add=True)
      return 0
    lax.fori_loop(0, NCHUNK, chunk_body, 0)
    plsc.subcore_barrier()

    for k in range(RPT // ZROWS):
      pltpu.sync_copy(agg_sh.at[pl.ds(base + k * ZROWS, ZROWS)], zbuf)
      pltpu.sync_copy(zbuf, agg_out.at[c, pl.ds(base + k * ZROWS, ZROWS)])
    pltpu.sync_copy(cnt_sh.at[pl.ds(base, RPT)], czbuf)
    pltpu.sync_copy(czbuf, cnt_out.at[c, pl.ds(base, RPT)])

  return sc_kernel(x, src3, dst3)


def _tc_finish_body(agg_ref, cnt_ref, x_ref, wl_ref, bl_ref, wr_ref,
                    g_ref, b_ref, out_ref):
  agg = agg_ref[0, :N] + agg_ref[1, :N]
  cnt = cnt_ref[0, :N] + cnt_ref[1, :N]
  mean = agg / jnp.clip(cnt, 1.0, None)[:, None]
  h = (jnp.dot(mean, wl_ref[...].T, preferred_element_type=jnp.float32)
       + bl_ref[...][None, :]
       + jnp.dot(x_ref[...], wr_ref[...].T, preferred_element_type=jnp.float32))
  h = jnp.maximum(h, 0.0)
  mu = jnp.mean(h, axis=0)
  var = jnp.mean((h - mu[None, :]) ** 2, axis=0)
  out_ref[...] = ((h - mu[None, :]) * lax.rsqrt(var + 1e-5)
                  * g_ref[...][None, :] + b_ref[...][None, :])


def kernel(x, edge_index, W_l, b_l, W_r, gamma, beta):
  src3 = edge_index[0].reshape(NW, NCHUNK, CHUNK)
  dst3 = edge_index[1].reshape(NW, NCHUNK, CHUNK)
  agg_p, cnt_p = _sc_aggregate(x, src3, dst3)
  return pl.pallas_call(
      _tc_finish_body,
      out_shape=jax.ShapeDtypeStruct((N, D), jnp.float32),
  )(agg_p, cnt_p, x, W_l, b_l, W_r, gamma, beta)


# speedup vs baseline: 13.8025x; 1.5702x over previous
import functools

import jax
import jax.numpy as jnp
from jax import lax
from jax.experimental import pallas as pl
from jax.experimental.pallas import tpu as pltpu
from jax.experimental.pallas import tpu_sc as plsc

N = 10000
E = 320000
D = 128

NC = 2
NS = 16
NW = NC * NS

E_PER_W = E // NW
CHUNK = 80
NCHUNK = E_PER_W // CHUNK
NPAD = 10240
RPT = NPAD // NS
ZROWS = 32


def _sc_aggregate(x, src3, dst3):
  mesh = plsc.VectorSubcoreMesh(core_axis_name="c", subcore_axis_name="s")

  @functools.partial(
      pl.kernel,
      out_type=(
          jax.ShapeDtypeStruct((NC, NPAD, D), jnp.float32),
          jax.ShapeDtypeStruct((NC, NPAD), jnp.float32),
      ),
      mesh=mesh,
      compiler_params=pltpu.CompilerParams(use_tc_tiling_on_sc=False),
      scratch_types=[
          pltpu.VMEM((NCHUNK, CHUNK), jnp.int32),
          pltpu.VMEM((NCHUNK, CHUNK), jnp.int32),
          pltpu.VMEM((CHUNK, D), jnp.float32),
          pltpu.VMEM((CHUNK, D), jnp.float32),
          pltpu.VMEM((CHUNK,), jnp.float32),
          pltpu.VMEM((ZROWS, D), jnp.float32),
          pltpu.VMEM((RPT,), jnp.float32),
          pltpu.VMEM_SHARED((NPAD, D), jnp.float32),
          pltpu.VMEM_SHARED((NPAD,), jnp.float32),
          pltpu.SemaphoreType.DMA,
          pltpu.SemaphoreType.DMA,
          pltpu.SemaphoreType.DMA,
          pltpu.SemaphoreType.DMA,
      ],
  )
  def sc_kernel(x_hbm, src_hbm, dst_hbm, agg_out, cnt_out,
                srcl, dstl, rows0, rows1, ones, zbuf, czbuf, agg_sh, cnt_sh,
                semg0, semg1, sems, semc):
    c = lax.axis_index("c")
    s = lax.axis_index("s")
    wid = s * NC + c

    def zrow(i, _):
      zbuf[i // 8, pl.ds((i % 8) * 16, 16)] = jnp.zeros((16,), jnp.float32)
      return 0
    lax.fori_loop(0, ZROWS * (D // 16), zrow, 0)

    def czrow(i, _):
      czbuf[pl.ds(i * 16, 16)] = jnp.zeros((16,), jnp.float32)
      return 0
    lax.fori_loop(0, RPT // 16, czrow, 0)

    def onerow(i, _):
      ones[pl.ds(i * 16, 16)] = jnp.ones((16,), jnp.float32)
      return 0
    lax.fori_loop(0, CHUNK // 16, onerow, 0)

    base = s * RPT
    for k in range(RPT // ZROWS):
      pltpu.sync_copy(zbuf, agg_sh.at[pl.ds(base + k * ZROWS, ZROWS)])
    pltpu.sync_copy(czbuf, cnt_sh.at[pl.ds(base, RPT)])
    plsc.subcore_barrier()

    pltpu.sync_copy(src_hbm.at[wid], srcl)
    pltpu.sync_copy(dst_hbm.at[wid], dstl)

    pltpu.async_copy(x_hbm.at[srcl.at[0]], rows0, semg0)
    pltpu.async_copy(x_hbm.at[srcl.at[1]], rows1, semg1)

    def step(i, rows_b, semg_b):
      pltpu.make_async_copy(x_hbm.at[srcl.at[i]], rows_b, semg_b).wait()
      sa = pltpu.async_copy(rows_b, agg_sh.at[dstl.at[i]], sems, add=True)
      sc = pltpu.async_copy(ones, cnt_sh.at[dstl.at[i]], semc, add=True)
      sa.wait()
      sc.wait()
      nxt = jnp.minimum(i + 2, NCHUNK - 1)
      pltpu.async_copy(x_hbm.at[srcl.at[nxt]], rows_b, semg_b)

    def chunk_body(i, _):
      @pl.when(i % 2 == 0)
      def _():
        step(i, rows0, semg0)
      @pl.when(i % 2 == 1)
      def _():
        step(i, rows1, semg1)
      return 0
    lax.fori_loop(0, NCHUNK, chunk_body, 0)
    pltpu.make_async_copy(x_hbm.at[srcl.at[NCHUNK - 1]], rows0, semg0).wait()
    pltpu.make_async_copy(x_hbm.at[srcl.at[NCHUNK - 1]], rows1, semg1).wait()
    plsc.subcore_barrier()

    for k in range(RPT // ZROWS):
      pltpu.sync_copy(agg_sh.at[pl.ds(base + k * ZROWS, ZROWS)], zbuf)
      pltpu.sync_copy(zbuf, agg_out.at[c, pl.ds(base + k * ZROWS, ZROWS)])
    pltpu.sync_copy(cnt_sh.at[pl.ds(base, RPT)], czbuf)
    pltpu.sync_copy(czbuf, cnt_out.at[c, pl.ds(base, RPT)])

  return sc_kernel(x, src3, dst3)


def _tc_finish_body(agg_ref, cnt_ref, x_ref, wl_ref, bl_ref, wr_ref,
                    g_ref, b_ref, out_ref):
  agg = agg_ref[0, :N] + agg_ref[1, :N]
  cnt = cnt_ref[0, :N] + cnt_ref[1, :N]
  mean = agg / jnp.clip(cnt, 1.0, None)[:, None]
  h = (jnp.dot(mean, wl_ref[...].T, preferred_element_type=jnp.float32)
       + bl_ref[...][None, :]
       + jnp.dot(x_ref[...], wr_ref[...].T, preferred_element_type=jnp.float32))
  h = jnp.maximum(h, 0.0)
  mu = jnp.mean(h, axis=0)
  var = jnp.mean((h - mu[None, :]) ** 2, axis=0)
  out_ref[...] = ((h - mu[None, :]) * lax.rsqrt(var + 1e-5)
                  * g_ref[...][None, :] + b_ref[...][None, :])


def kernel(x, edge_index, W_l, b_l, W_r, gamma, beta):
  src3 = edge_index[0].reshape(NW, NCHUNK, CHUNK)
  dst3 = edge_index[1].reshape(NW, NCHUNK, CHUNK)
  agg_p, cnt_p = _sc_aggregate(x, src3, dst3)
  return pl.pallas_call(
      _tc_finish_body,
      out_shape=jax.ShapeDtypeStruct((N, D), jnp.float32),
  )(agg_p, cnt_p, x, W_l, b_l, W_r, gamma, beta)
